# baseline (speedup 1.0000x reference)
import functools

import jax
import jax.numpy as jnp
from jax import lax
from jax.experimental import pallas as pl
from jax.experimental.pallas import tpu as pltpu
from jax.experimental.pallas import tpu_sc as plsc

BATCH = 16384
EMB = 64
NC = 2
NS = 16
NW = NC * NS
BPW = BATCH // NW
WAVE = 256
NWAVE = BPW // WAVE
CHUNK = 128
LANES = 16

_mesh = plsc.VectorSubcoreMesh(core_axis_name="c", subcore_axis_name="s")


@functools.partial(
    pl.kernel,
    out_type=jax.ShapeDtypeStruct((NW, BPW), jnp.float32),
    mesh=_mesh,
    compiler_params=pltpu.CompilerParams(
        needs_layout_passes=False,
    ),
    scratch_types=[
        pltpu.VMEM((BPW,), jnp.int32),
        pltpu.VMEM((BPW,), jnp.int32),
        pltpu.VMEM((BPW,), jnp.int32),
        pltpu.VMEM((BPW,), jnp.int32),
        pltpu.VMEM((WAVE, 128), jnp.float32),
        pltpu.VMEM((WAVE, 128), jnp.float32),
        pltpu.VMEM((BPW,), jnp.float32),
        pltpu.SemaphoreType.DMA,
    ],
)
def _mf_dot_kernel(u_id_hbm, i_id_hbm, utab_hbm, itab_hbm, out_hbm,
                   uid_v, iid_v, ukey_v, ikey_v, udat_v, idat_v, out_v, sem):
    wid = lax.axis_index("s") * NC + lax.axis_index("c")

    pltpu.sync_copy(u_id_hbm.at[wid], uid_v)
    pltpu.sync_copy(i_id_hbm.at[wid], iid_v)

    for o in range(BPW // LANES):
        sl = pl.ds(o * LANES, LANES)
        ukey_v[sl] = lax.shift_right_logical(uid_v[sl], 1)
        ikey_v[sl] = lax.shift_right_logical(iid_v[sl], 1)

    lane = lax.iota(jnp.int32, LANES)

    def wave_body(w, carry):
        base = w * WAVE
        copies = []
        for c in range(WAVE // CHUNK):
            sl = pl.ds(base + c * CHUNK, CHUNK)
            dst = pl.ds(c * CHUNK, CHUNK)
            copies.append(
                pltpu.async_copy(utab_hbm.at[ukey_v.at[sl]],
                                 udat_v.at[dst], sem))
            copies.append(
                pltpu.async_copy(itab_hbm.at[ikey_v.at[sl]],
                                 idat_v.at[dst], sem))
        for cp in copies:
            cp.wait()

        def group_body(o, carry2):
            rows = o * LANES + lane
            sl = pl.ds(base + o * LANES, LANES)
            upar = (uid_v[sl] & 1) * EMB
            ipar = (iid_v[sl] & 1) * EMB
            acc = jnp.zeros((LANES,), jnp.float32)
            for d in range(EMB):
                rot = (d + lane) & (EMB - 1)
                u = plsc.load_gather(udat_v, [rows, upar + rot])
                v = plsc.load_gather(idat_v, [rows, ipar + rot])
                acc = acc + u * v
            out_v[sl] = acc
            return carry2

        lax.fori_loop(0, WAVE // LANES, group_body, 0)
        return carry

    lax.fori_loop(0, NWAVE, wave_body, 0)

    pltpu.sync_copy(out_v, out_hbm.at[wid])


def kernel(u_id, i_id, user_emb, item_emb):
    u2 = u_id.astype(jnp.int32).reshape(NW, BPW)
    i2 = i_id.astype(jnp.int32).reshape(NW, BPW)
    eye = jnp.eye(128, dtype=jnp.float32)
    utab = user_emb.reshape(500000, 128) @ eye
    itab = item_emb.reshape(500000, 128) @ eye
    out = _mf_dot_kernel(u2, i2, utab, itab)
    return out.reshape(BATCH)


# speedup vs baseline: 1.3160x; 1.3160x over previous
import functools

import jax
import jax.numpy as jnp
from jax import lax
from jax.experimental import pallas as pl
from jax.experimental.pallas import tpu as pltpu
from jax.experimental.pallas import tpu_sc as plsc

BATCH = 16384
EMB = 64
NC = 2
NS = 16
NW = NC * NS
BPW = BATCH // NW
CHUNK = 128
NCHUNK = BPW // CHUNK
LANES = 16
NGROUP = BPW // LANES

_mesh = plsc.VectorSubcoreMesh(core_axis_name="c", subcore_axis_name="s")


@functools.partial(
    pl.kernel,
    out_type=jax.ShapeDtypeStruct((NW, BPW), jnp.float32),
    mesh=_mesh,
    compiler_params=pltpu.CompilerParams(
        needs_layout_passes=False,
        use_tc_tiling_on_sc=False,
    ),
    scratch_types=[
        pltpu.VMEM((NCHUNK, CHUNK), jnp.int32),
        pltpu.VMEM((NCHUNK, CHUNK), jnp.int32),
        pltpu.VMEM((BPW, EMB), jnp.float32),
        pltpu.VMEM((BPW, EMB), jnp.float32),
        pltpu.VMEM((BPW,), jnp.float32),
        pltpu.SemaphoreType.DMA,
    ],
)
def _mf_dot_kernel(u_id_hbm, i_id_hbm, user_hbm, item_hbm, out_hbm,
                   uid_v, iid_v, urows_v, irows_v, out_v, sem):
    wid = lax.axis_index("s") * NC + lax.axis_index("c")

    pltpu.sync_copy(u_id_hbm.at[pl.ds(wid * NCHUNK, NCHUNK)], uid_v)
    pltpu.sync_copy(i_id_hbm.at[pl.ds(wid * NCHUNK, NCHUNK)], iid_v)

    copies = []
    for c in range(NCHUNK):
        dst = urows_v.at[pl.ds(c * CHUNK, CHUNK)]
        copies.append(pltpu.async_copy(user_hbm.at[uid_v.at[c]], dst, sem))
        dst = irows_v.at[pl.ds(c * CHUNK, CHUNK)]
        copies.append(pltpu.async_copy(item_hbm.at[iid_v.at[c]], dst, sem))
    for cp in copies:
        cp.wait()

    lane = lax.iota(jnp.int32, LANES)

    def group_body(g, carry):
        rows = g * LANES + lane
        acc = jnp.zeros((LANES,), jnp.float32)
        for d in range(EMB):
            cols = (lane + d) & (EMB - 1)
            u = plsc.load_gather(urows_v, [rows, cols])
            v = plsc.load_gather(irows_v, [rows, cols])
            acc = acc + u * v
        out_v[pl.ds(g * LANES, LANES)] = acc
        return carry

    lax.fori_loop(0, NGROUP, group_body, 0)

    pltpu.sync_copy(out_v, out_hbm.at[wid])


def kernel(u_id, i_id, user_emb, item_emb):
    u2 = u_id.astype(jnp.int32).reshape(NW * NCHUNK, CHUNK)
    i2 = i_id.astype(jnp.int32).reshape(NW * NCHUNK, CHUNK)
    out = _mf_dot_kernel(u2, i2, user_emb, item_emb)
    return out.reshape(BATCH)
